# baseline (speedup 1.0000x reference)
import functools

import jax
import jax.numpy as jnp
from jax import lax
from jax.experimental import pallas as pl
from jax.experimental.pallas import tpu as pltpu
from jax.experimental.pallas import tpu_sc as plsc

NUM_CORES = 2
NUM_SUBCORES = 16
NW = NUM_CORES * NUM_SUBCORES

CHUNK = 128


def _gather_kernel_body(n_chunks, table_hbm, idx_hbm, out_hbm, idx_v, rows_v, gsem):
    wid = lax.axis_index("s") * NUM_CORES + lax.axis_index("c")
    pltpu.sync_copy(idx_hbm.at[wid], idx_v)
    out_base = wid * (n_chunks * CHUNK)

    def body(c, _):
        pltpu.async_copy(table_hbm.at[idx_v.at[c]], rows_v, gsem).wait()
        pltpu.sync_copy(rows_v, out_hbm.at[pl.ds(out_base + c * CHUNK, CHUNK)])
        return 0

    lax.fori_loop(0, n_chunks, body, 0)


def _make_gather(v, d, b_total):
    assert b_total % (NW * CHUNK) == 0
    n_chunks = b_total // (NW * CHUNK)
    mesh = plsc.VectorSubcoreMesh(
        core_axis_name="c", subcore_axis_name="s",
        num_cores=NUM_CORES, num_subcores=NUM_SUBCORES)
    return pl.kernel(
        functools.partial(_gather_kernel_body, n_chunks),
        out_type=jax.ShapeDtypeStruct((b_total, d), jnp.float32),
        mesh=mesh,
        scratch_types=[
            pltpu.VMEM((n_chunks, CHUNK), jnp.int32),
            pltpu.VMEM((CHUNK, d), jnp.float32),
            pltpu.SemaphoreType.DMA,
        ],
        compiler_params=pltpu.CompilerParams(use_tc_tiling_on_sc=False),
    )


def kernel(vocab_ids, table):
    bsz, seq = vocab_ids.shape
    v, d = table.shape
    b_total = bsz * seq
    idx = vocab_ids.reshape(NW, b_total // (NW * CHUNK), CHUNK).astype(jnp.int32)
    out = _make_gather(v, d, b_total)(table, idx)
    return out.reshape(bsz, seq, d)


# speedup vs baseline: 1.1201x; 1.1201x over previous
import functools

import jax
import jax.numpy as jnp
from jax import lax
from jax.experimental import pallas as pl
from jax.experimental.pallas import tpu as pltpu
from jax.experimental.pallas import tpu_sc as plsc

NUM_CORES = 2
NUM_SUBCORES = 16
NW = NUM_CORES * NUM_SUBCORES

CHUNK = 128
NBUF = 8
LA = NBUF - 2


def _gather_kernel_body(n_chunks, table_hbm, idx_hbm, out_hbm, *scratch):
    idx_v = scratch[0]
    rows = scratch[1:1 + NBUF]
    gsems = scratch[1 + NBUF:1 + 2 * NBUF]
    osems = scratch[1 + 2 * NBUF:1 + 3 * NBUF]

    wid = lax.axis_index("s") * NUM_CORES + lax.axis_index("c")
    pltpu.sync_copy(idx_hbm.at[wid], idx_v)
    out_base = wid * (n_chunks * CHUNK)

    def start_gather(c, b):
        pltpu.async_copy(table_hbm.at[idx_v.at[c]], rows[b], gsems[b])

    def wait_gather(c, b):
        pltpu.make_async_copy(table_hbm.at[idx_v.at[c]], rows[b], gsems[b]).wait()

    def start_write(c, b):
        pltpu.async_copy(rows[b], out_hbm.at[pl.ds(out_base + c * CHUNK, CHUNK)],
                         osems[b])

    def wait_write(b):
        pltpu.make_async_copy(rows[b], out_hbm.at[pl.ds(out_base, CHUNK)],
                              osems[b]).wait()

    n_groups = n_chunks // NBUF

    for c in range(LA):
        start_gather(c, c)

    for c in range(NBUF):
        b = c
        wait_gather(c, b)
        start_write(c, b)
        bf = (b + LA) % NBUF
        if c >= 2:
            wait_write(bf)
        start_gather(c + LA, bf)

    def body(g, _):
        for b in range(NBUF):
            c = g * NBUF + b
            wait_gather(c, b)
            start_write(c, b)
            bf = (b + LA) % NBUF
            wait_write(bf)
            start_gather(c + LA, bf)
        return 0

    lax.fori_loop(1, n_groups - 1, body, 0)

    for c in range((n_groups - 1) * NBUF, n_chunks):
        b = c % NBUF
        wait_gather(c, b)
        start_write(c, b)
        if c + LA < n_chunks:
            bf = (b + LA) % NBUF
            wait_write(bf)
            start_gather(c + LA, bf)

    for b in range(NBUF):
        wait_write(b)


def _make_gather(v, d, b_total):
    assert b_total % (NW * CHUNK) == 0
    n_chunks = b_total // (NW * CHUNK)
    assert n_chunks % NBUF == 0 and n_chunks // NBUF >= 2
    mesh = plsc.VectorSubcoreMesh(
        core_axis_name="c", subcore_axis_name="s",
        num_cores=NUM_CORES, num_subcores=NUM_SUBCORES)
    return pl.kernel(
        functools.partial(_gather_kernel_body, n_chunks),
        out_type=jax.ShapeDtypeStruct((b_total, d), jnp.float32),
        mesh=mesh,
        scratch_types=(
            [pltpu.VMEM((n_chunks, CHUNK), jnp.int32)]
            + [pltpu.VMEM((CHUNK, d), jnp.float32) for _ in range(NBUF)]
            + [pltpu.SemaphoreType.DMA for _ in range(2 * NBUF)]
        ),
        compiler_params=pltpu.CompilerParams(use_tc_tiling_on_sc=False),
    )


def kernel(vocab_ids, table):
    bsz, seq = vocab_ids.shape
    v, d = table.shape
    b_total = bsz * seq
    idx = vocab_ids.reshape(NW, b_total // (NW * CHUNK), CHUNK).astype(jnp.int32)
    out = _make_gather(v, d, b_total)(table, idx)
    return out.reshape(bsz, seq, d)


# speedup vs baseline: 1.4850x; 1.3258x over previous
import functools

import jax
import jax.numpy as jnp
from jax import lax
from jax.experimental import pallas as pl
from jax.experimental.pallas import tpu as pltpu
from jax.experimental.pallas import tpu_sc as plsc

NUM_CORES = 2
NUM_SUBCORES = 16
NW = NUM_CORES * NUM_SUBCORES

CHUNK = 128
NBUF = 8
LA = NBUF - 2


def _gather_kernel_body(n_chunks, table_hbm, idx_hbm, out_hbm, *scratch):
    idx_v = scratch[0]
    rows = scratch[1:1 + NBUF]
    gsems = scratch[1 + NBUF:1 + 2 * NBUF]
    osems = scratch[1 + 2 * NBUF:1 + 3 * NBUF]

    wid = lax.axis_index("s") * NUM_CORES + lax.axis_index("c")
    pltpu.sync_copy(idx_hbm.at[wid], idx_v)
    out_base = wid * (n_chunks * CHUNK)

    def start_gather(c, b):
        pltpu.async_copy(table_hbm.at[idx_v.at[c]], rows[b], gsems[b])

    def wait_gather(c, b):
        pltpu.make_async_copy(table_hbm.at[idx_v.at[c]], rows[b], gsems[b]).wait()

    def start_write(c, b):
        pltpu.async_copy(
            rows[b],
            out_hbm.at[pl.ds(out_base + c * CHUNK, CHUNK), pl.ds(0, 64)],
            osems[b])

    def wait_write(b):
        pltpu.make_async_copy(
            rows[b], out_hbm.at[pl.ds(out_base, CHUNK), pl.ds(0, 64)],
            osems[b]).wait()

    n_groups = n_chunks // NBUF

    for c in range(LA):
        start_gather(c, c)

    for c in range(NBUF):
        b = c
        wait_gather(c, b)
        start_write(c, b)
        bf = (b + LA) % NBUF
        if c >= 2:
            wait_write(bf)
        start_gather(c + LA, bf)

    def body(g, _):
        for b in range(NBUF):
            c = g * NBUF + b
            wait_gather(c, b)
            start_write(c, b)
            bf = (b + LA) % NBUF
            wait_write(bf)
            start_gather(c + LA, bf)
        return 0

    lax.fori_loop(1, n_groups - 1, body, 0)

    for c in range((n_groups - 1) * NBUF, n_chunks):
        b = c % NBUF
        wait_gather(c, b)
        start_write(c, b)
        if c + LA < n_chunks:
            bf = (b + LA) % NBUF
            wait_write(bf)
            start_gather(c + LA, bf)

    for b in range(NBUF):
        wait_write(b)


def _make_gather(v, d, b_total):
    assert b_total % (NW * CHUNK) == 0
    n_chunks = b_total // (NW * CHUNK)
    assert n_chunks % NBUF == 0 and n_chunks // NBUF >= 2
    mesh = plsc.VectorSubcoreMesh(
        core_axis_name="c", subcore_axis_name="s",
        num_cores=NUM_CORES, num_subcores=NUM_SUBCORES)
    return pl.kernel(
        functools.partial(_gather_kernel_body, n_chunks),
        out_type=jax.ShapeDtypeStruct((b_total, 128), jnp.float32),
        mesh=mesh,
        scratch_types=(
            [pltpu.VMEM((n_chunks, CHUNK), jnp.int32)]
            + [pltpu.VMEM((CHUNK, d), jnp.float32) for _ in range(NBUF)]
            + [pltpu.SemaphoreType.DMA for _ in range(2 * NBUF)]
        ),
        compiler_params=pltpu.CompilerParams(use_tc_tiling_on_sc=False),
    )


def kernel(vocab_ids, table):
    bsz, seq = vocab_ids.shape
    v, d = table.shape
    b_total = bsz * seq
    idx = vocab_ids.reshape(NW, b_total // (NW * CHUNK), CHUNK).astype(jnp.int32)
    out = _make_gather(v, d, b_total)(table, idx)
    return out.reshape(bsz, seq, 128)[:, :, :d]
